# baseline (speedup 1.0000x reference)
import functools

import jax
import jax.numpy as jnp
from jax import lax
from jax.experimental import pallas as pl
from jax.experimental.pallas import tpu as pltpu
from jax.experimental.pallas import tpu_sc as plsc

BATCH = 16384
D = 128
INDEX_OFFSET = 100
NC = 2
NS = 16
NW = NC * NS
CHUNK = 64
ROWS_PER_W = BATCH // (NW * CHUNK)


def kernel(x, table):
    x3 = x.reshape(NW, ROWS_PER_W, CHUNK)
    mesh = plsc.VectorSubcoreMesh(core_axis_name="c", subcore_axis_name="s")

    @functools.partial(
        pl.kernel,
        mesh=mesh,
        out_type=jax.ShapeDtypeStruct((NW, ROWS_PER_W, CHUNK, D), jnp.float32),
        scratch_types=[
            pltpu.VMEM((ROWS_PER_W, CHUNK), jnp.int32),
            pltpu.VMEM((ROWS_PER_W, CHUNK, D), jnp.float32),
        ]
        + [pltpu.SemaphoreType.DMA] * ROWS_PER_W
        + [pltpu.SemaphoreType.DMA],
    )
    def emb(x_hbm, table_hbm, out_hbm, idx_v, rows_v, *sems):
        gsems, wsem = sems[:ROWS_PER_W], sems[ROWS_PER_W]
        wid = lax.axis_index("s") * NC + lax.axis_index("c")
        pltpu.sync_copy(x_hbm.at[wid], idx_v)
        for i in range(ROWS_PER_W):
            row = idx_v.at[i]
            for j in range(CHUNK // 16):
                s = pl.ds(j * 16, 16)
                row[s] = row[s] + INDEX_OFFSET
        gathers = [
            pltpu.async_copy(table_hbm.at[idx_v.at[i]], rows_v.at[i], gsems[i])
            for i in range(ROWS_PER_W)
        ]
        writes = []
        for i in range(ROWS_PER_W):
            gathers[i].wait()
            writes.append(
                pltpu.async_copy(rows_v.at[i], out_hbm.at[wid, i], wsem)
            )
        for w in writes:
            w.wait()

    return emb(x3, table).reshape(BATCH, D)


# speedup vs baseline: 1.0625x; 1.0625x over previous
import functools

import jax
import jax.numpy as jnp
from jax import lax
from jax.experimental import pallas as pl
from jax.experimental.pallas import tpu as pltpu
from jax.experimental.pallas import tpu_sc as plsc

BATCH = 16384
D = 128
INDEX_OFFSET = 100
NC = 2
NS = 16
NW = NC * NS
CHUNK = 128
ROWS_PER_W = BATCH // (NW * CHUNK)


def kernel(x, table):
    x3 = x.reshape(NW, ROWS_PER_W, CHUNK)
    mesh = plsc.VectorSubcoreMesh(core_axis_name="c", subcore_axis_name="s")

    @functools.partial(
        pl.kernel,
        mesh=mesh,
        out_type=jax.ShapeDtypeStruct((NW, ROWS_PER_W, CHUNK, D), jnp.float32),
        scratch_types=[
            pltpu.VMEM((ROWS_PER_W, CHUNK), jnp.int32),
            pltpu.VMEM((ROWS_PER_W, CHUNK, D), jnp.float32),
        ]
        + [pltpu.SemaphoreType.DMA] * ROWS_PER_W
        + [pltpu.SemaphoreType.DMA],
    )
    def emb(x_hbm, table_hbm, out_hbm, idx_v, rows_v, *sems):
        gsems, wsem = sems[:ROWS_PER_W], sems[ROWS_PER_W]
        wid = lax.axis_index("s") * NC + lax.axis_index("c")
        pltpu.sync_copy(x_hbm.at[wid], idx_v)
        for i in range(ROWS_PER_W):
            row = idx_v.at[i]
            for j in range(CHUNK // 16):
                s = pl.ds(j * 16, 16)
                row[s] = row[s] + INDEX_OFFSET
        gathers = [
            pltpu.async_copy(table_hbm.at[idx_v.at[i]], rows_v.at[i], gsems[i])
            for i in range(ROWS_PER_W)
        ]
        writes = []
        for i in range(ROWS_PER_W):
            gathers[i].wait()
            writes.append(
                pltpu.async_copy(rows_v.at[i], out_hbm.at[wid, i], wsem)
            )
        for w in writes:
            w.wait()

    return emb(x3, table).reshape(BATCH, D)


# speedup vs baseline: 1.7498x; 1.6469x over previous
import functools

import jax
import jax.numpy as jnp
from jax import lax
from jax.experimental import pallas as pl
from jax.experimental.pallas import tpu as pltpu
from jax.experimental.pallas import tpu_sc as plsc

BATCH = 16384
D = 128
INDEX_OFFSET = 100
NC = 2
NS = 16
NW = NC * NS
CHUNK = 128
ROWS_PER_W = BATCH // (NW * CHUNK)


def kernel(x, table):
    x3 = x.reshape(NW, ROWS_PER_W, CHUNK)
    mesh = plsc.VectorSubcoreMesh(core_axis_name="c", subcore_axis_name="s")

    @functools.partial(
        pl.kernel,
        mesh=mesh,
        out_type=jax.ShapeDtypeStruct((NW, ROWS_PER_W, CHUNK, D), jnp.float32),
        scratch_types=[
            pltpu.VMEM((ROWS_PER_W, CHUNK), jnp.int32),
            pltpu.VMEM((ROWS_PER_W, CHUNK, D), jnp.float32),
            pltpu.VMEM_SHARED((201, D), jnp.float32),
            pltpu.SemaphoreType.DMA,
        ],
    )
    def emb(x_hbm, table_hbm, out_hbm, idx_v, rows_v, tab_sp, sem):
        sid = lax.axis_index("s")
        wid = sid * NC + lax.axis_index("c")

        @pl.when(sid == 0)
        def _():
            pltpu.sync_copy(table_hbm, tab_sp)

        pltpu.sync_copy(x_hbm.at[wid], idx_v)
        for i in range(ROWS_PER_W):
            row = idx_v.at[i]
            for j in range(CHUNK // 16):
                s = pl.ds(j * 16, 16)
                row[s] = row[s] + INDEX_OFFSET
        plsc.subcore_barrier()
        copies = [
            pltpu.async_copy(tab_sp.at[idx_v.at[i]], rows_v.at[i], sem)
            for i in range(ROWS_PER_W)
        ]
        for c in copies:
            c.wait()
        pltpu.sync_copy(rows_v, out_hbm.at[wid])

    return emb(x3, table).reshape(BATCH, D)


# speedup vs baseline: 1.8086x; 1.0336x over previous
import functools

import jax
import jax.numpy as jnp
from jax import lax
from jax.experimental import pallas as pl
from jax.experimental.pallas import tpu as pltpu
from jax.experimental.pallas import tpu_sc as plsc

BATCH = 16384
D = 128
INDEX_OFFSET = 100
NC = 2
NS = 16
NW = NC * NS
CHUNK = 128
ROWS_PER_W = BATCH // (NW * CHUNK)


def kernel(x, table):
    x3 = x.reshape(NW, ROWS_PER_W, CHUNK)
    mesh = plsc.VectorSubcoreMesh(core_axis_name="c", subcore_axis_name="s")

    @functools.partial(
        pl.kernel,
        mesh=mesh,
        out_type=jax.ShapeDtypeStruct((NW, ROWS_PER_W, CHUNK, D), jnp.float32),
        scratch_types=[
            pltpu.VMEM((ROWS_PER_W, CHUNK), jnp.int32),
            pltpu.VMEM((ROWS_PER_W, CHUNK, D), jnp.float32),
            pltpu.VMEM_SHARED((201, D), jnp.float32),
        ]
        + [pltpu.SemaphoreType.DMA] * ROWS_PER_W
        + [pltpu.SemaphoreType.DMA],
    )
    def emb(x_hbm, table_hbm, out_hbm, idx_v, rows_v, tab_sp, *sems):
        gsems, wsem = sems[:ROWS_PER_W], sems[ROWS_PER_W]
        sid = lax.axis_index("s")
        wid = sid * NC + lax.axis_index("c")

        @pl.when(sid == 0)
        def _():
            pltpu.sync_copy(table_hbm, tab_sp)

        pltpu.sync_copy(x_hbm.at[wid], idx_v)
        for i in range(ROWS_PER_W):
            row = idx_v.at[i]
            for j in range(CHUNK // 16):
                s = pl.ds(j * 16, 16)
                row[s] = row[s] + INDEX_OFFSET
        plsc.subcore_barrier()
        gathers = [
            pltpu.async_copy(tab_sp.at[idx_v.at[i]], rows_v.at[i], gsems[i])
            for i in range(ROWS_PER_W)
        ]
        writes = []
        for i in range(ROWS_PER_W):
            gathers[i].wait()
            writes.append(
                pltpu.async_copy(rows_v.at[i], out_hbm.at[wid, i], wsem)
            )
        for w in writes:
            w.wait()

    return emb(x3, table).reshape(BATCH, D)
